# baseline (speedup 1.0000x reference)
import functools

import jax
import jax.numpy as jnp
from jax import lax
from jax.experimental import pallas as pl
from jax.experimental.pallas import tpu as pltpu
from jax.experimental.pallas import tpu_sc as plsc

VOCAB = 1000000
D = 64
B = 4096
L = 200
HID = 256
NCLS = 10

NC = 2
NS = 16
NW = NC * NS

LP = 208
LH = LP // 2
BPW = B // NW

_INV_L = 1.0 / L


def _pool_body(tok0, tok1, tok2, emb0, emb1, emb2, out,
               idx_v, rows_v, pooled_v, sem0, sem1):
    wid = lax.axis_index("s") * NC + lax.axis_index("c")
    base = wid * BPW
    sems = (sem0, sem1)

    def gather(table, b, buf):
        c0 = pltpu.async_copy(table.at[idx_v.at[b, 0]],
                              rows_v.at[buf, pl.ds(0, LH)], sems[buf])
        c1 = pltpu.async_copy(table.at[idx_v.at[b, 1]],
                              rows_v.at[buf, pl.ds(LH, LH)], sems[buf])
        return c0, c1

    def wait(table, b, buf):
        pltpu.make_async_copy(table.at[idx_v.at[b, 0]],
                              rows_v.at[buf, pl.ds(0, LH)], sems[buf]).wait()
        pltpu.make_async_copy(table.at[idx_v.at[b, 1]],
                              rows_v.at[buf, pl.ds(LH, LH)], sems[buf]).wait()

    def accum_store(b, buf):
        rbuf = rows_v.at[buf]

        def body(i, carry):
            a0, a1, a2, a3 = carry
            return (a0 + rbuf[i, 0:16], a1 + rbuf[i, 16:32],
                    a2 + rbuf[i, 32:48], a3 + rbuf[i, 48:64])

        z = jnp.zeros((16,), jnp.float32)
        a0, a1, a2, a3 = lax.fori_loop(0, L, body, (z, z, z, z))
        pooled_v[b, 0:16] = a0 * _INV_L
        pooled_v[b, 16:32] = a1 * _INV_L
        pooled_v[b, 32:48] = a2 * _INV_L
        pooled_v[b, 48:64] = a3 * _INV_L

    for t, (tok, table) in enumerate(((tok0, emb0), (tok1, emb1), (tok2, emb2))):
        pltpu.sync_copy(tok.at[pl.ds(base, BPW)], idx_v)
        gather(table, 0, 0)

        def step(bb, table=table):
            for u in range(2):
                cur = u
                b = bb + u
                gather(table, b + 1, 1 - cur)
                wait(table, b, cur)
                accum_store(b, cur)

        pl.loop(0, BPW - 2, step=2)(step)
        gather(table, BPW - 1, 1)
        wait(table, BPW - 2, 0)
        accum_store(BPW - 2, 0)
        wait(table, BPW - 1, 1)
        accum_store(BPW - 1, 1)

        pltpu.sync_copy(pooled_v, out.at[t, pl.ds(base, BPW)])


def _pooled_means(tok0p, tok1p, tok2p, emb0, emb1, emb2):
    mesh = plsc.VectorSubcoreMesh(core_axis_name="c", subcore_axis_name="s",
                                  num_cores=NC, num_subcores=NS)
    return pl.kernel(
        _pool_body,
        out_type=jax.ShapeDtypeStruct((3, B, D), jnp.float32),
        mesh=mesh,
        compiler_params=pltpu.CompilerParams(use_tc_tiling_on_sc=False),
        scratch_types=[
            pltpu.VMEM((BPW, 2, LH), jnp.int32),
            pltpu.VMEM((2, LP, D), jnp.float32),
            pltpu.VMEM((BPW, D), jnp.float32),
            pltpu.SemaphoreType.DMA,
            pltpu.SemaphoreType.DMA,
        ],
    )(tok0p, tok1p, tok2p, emb0, emb1, emb2)


def _mlp_body(x_ref, w1_ref, b1_ref, w2_ref, b2_ref, o_ref):
    h = (jnp.dot(x_ref[0], w1_ref[0], preferred_element_type=jnp.float32) +
         jnp.dot(x_ref[1], w1_ref[1], preferred_element_type=jnp.float32) +
         jnp.dot(x_ref[2], w1_ref[2], preferred_element_type=jnp.float32))
    h = jnp.maximum(h + b1_ref[...], 0.0)
    o_ref[...] = jnp.dot(h, w2_ref[...],
                         preferred_element_type=jnp.float32) + b2_ref[...]


def _mlp(pooled, W1, b1, W2, b2):
    BB = 1024
    grid = (B // BB,)
    return pl.pallas_call(
        _mlp_body,
        grid=grid,
        in_specs=[
            pl.BlockSpec((3, BB, D), lambda i: (0, i, 0)),
            pl.BlockSpec((3, D, HID), lambda i: (0, 0, 0)),
            pl.BlockSpec((1, HID), lambda i: (0, 0)),
            pl.BlockSpec((HID, NCLS), lambda i: (0, 0)),
            pl.BlockSpec((1, NCLS), lambda i: (0, 0)),
        ],
        out_specs=pl.BlockSpec((BB, NCLS), lambda i: (i, 0)),
        out_shape=jax.ShapeDtypeStruct((B, NCLS), jnp.float32),
    )(pooled, W1.reshape(3, D, HID), b1.reshape(1, HID), W2, b2.reshape(1, NCLS))


def kernel(tokens_0, tokens_1, tokens_2, emb_uni, emb_bi, emb_tri,
           W1, b1, W2, b2):
    pad = jnp.zeros((B, LP - L), jnp.int32)
    tok0p = jnp.concatenate((tokens_0, pad), axis=1).reshape(B, 2, LH)
    tok1p = jnp.concatenate((tokens_1, pad), axis=1).reshape(B, 2, LH)
    tok2p = jnp.concatenate((tokens_2, pad), axis=1).reshape(B, 2, LH)
    pooled = _pooled_means(tok0p, tok1p, tok2p, emb_uni, emb_bi, emb_tri)
    return _mlp(pooled, W1, b1, W2, b2)


# speedup vs baseline: 2.0707x; 2.0707x over previous
import functools

import jax
import jax.numpy as jnp
from jax import lax
from jax.experimental import pallas as pl
from jax.experimental.pallas import tpu as pltpu
from jax.experimental.pallas import tpu_sc as plsc

VOCAB = 1000000
D = 64
B = 4096
L = 200
HID = 256
NCLS = 10

NC = 2
NS = 16
NW = NC * NS

LA = 104
LB = L - LA
BPW = B // NW
NBUF = 4

_INV_L = 1.0 / L


def _pool_body(tok0, tok1, tok2, emb0, emb1, emb2, out,
               idx_v, rows_v, pooled_v, *sems):
    wid = lax.axis_index("s") * NC + lax.axis_index("c")
    base = wid * BPW

    def gather(table, b, buf):
        pltpu.async_copy(table.at[idx_v.at[b, pl.ds(0, LA)]],
                         rows_v.at[buf, pl.ds(0, LA)], sems[buf])
        pltpu.async_copy(table.at[idx_v.at[b, pl.ds(LA, LB)]],
                         rows_v.at[buf, pl.ds(LA, LB)], sems[buf])

    def wait(table, b, buf):
        pltpu.make_async_copy(table.at[idx_v.at[b, pl.ds(0, LA)]],
                              rows_v.at[buf, pl.ds(0, LA)], sems[buf]).wait()
        pltpu.make_async_copy(table.at[idx_v.at[b, pl.ds(LA, LB)]],
                              rows_v.at[buf, pl.ds(LA, LB)], sems[buf]).wait()

    def accum_store(b, buf):
        rbuf = rows_v.at[buf]

        def body(k, carry):
            a0, a1, a2, a3 = carry
            i = k * 8
            for j in range(8):
                a0 = a0 + rbuf[i + j, 0:16]
                a1 = a1 + rbuf[i + j, 16:32]
                a2 = a2 + rbuf[i + j, 32:48]
                a3 = a3 + rbuf[i + j, 48:64]
            return (a0, a1, a2, a3)

        z = jnp.zeros((16,), jnp.float32)
        a0, a1, a2, a3 = lax.fori_loop(0, L // 8, body, (z, z, z, z))
        pooled_v[b, 0:16] = a0 * _INV_L
        pooled_v[b, 16:32] = a1 * _INV_L
        pooled_v[b, 32:48] = a2 * _INV_L
        pooled_v[b, 48:64] = a3 * _INV_L

    MAIN = BPW - NBUF
    for t, (tok, table) in enumerate(((tok0, emb0), (tok1, emb1), (tok2, emb2))):
        pltpu.sync_copy(tok.at[pl.ds(base, BPW)], idx_v)
        for b in range(NBUF - 1):
            gather(table, b, b)

        def step(bb, table=table):
            for u in range(NBUF):
                b = bb + u
                gather(table, b + NBUF - 1, (u + NBUF - 1) % NBUF)
                wait(table, b, u)
                accum_store(b, u)

        pl.loop(0, MAIN, step=NBUF)(step)
        gather(table, BPW - 1, NBUF - 1)
        for u in range(NBUF):
            b = MAIN + u
            wait(table, b, u)
            accum_store(b, u)

        pltpu.sync_copy(pooled_v, out.at[t, pl.ds(base, BPW)])


def _pooled_means(tokens_0, tokens_1, tokens_2, emb0, emb1, emb2):
    mesh = plsc.VectorSubcoreMesh(core_axis_name="c", subcore_axis_name="s",
                                  num_cores=NC, num_subcores=NS)
    return pl.kernel(
        _pool_body,
        out_type=jax.ShapeDtypeStruct((3, B, D), jnp.float32),
        mesh=mesh,
        compiler_params=pltpu.CompilerParams(use_tc_tiling_on_sc=False),
        scratch_types=[
            pltpu.VMEM((BPW, L), jnp.int32),
            pltpu.VMEM((NBUF, L, D), jnp.float32),
            pltpu.VMEM((BPW, D), jnp.float32),
        ] + [pltpu.SemaphoreType.DMA] * NBUF,
    )(tokens_0, tokens_1, tokens_2, emb0, emb1, emb2)


def _mlp_body(x_ref, w1_ref, b1_ref, w2_ref, b2_ref, o_ref):
    h = (jnp.dot(x_ref[0], w1_ref[0], preferred_element_type=jnp.float32) +
         jnp.dot(x_ref[1], w1_ref[1], preferred_element_type=jnp.float32) +
         jnp.dot(x_ref[2], w1_ref[2], preferred_element_type=jnp.float32))
    h = jnp.maximum(h + b1_ref[...], 0.0)
    o_ref[...] = jnp.dot(h, w2_ref[...],
                         preferred_element_type=jnp.float32) + b2_ref[...]


def _mlp(pooled, W1, b1, W2, b2):
    BB = 1024
    grid = (B // BB,)
    return pl.pallas_call(
        _mlp_body,
        grid=grid,
        in_specs=[
            pl.BlockSpec((3, BB, D), lambda i: (0, i, 0)),
            pl.BlockSpec((3, D, HID), lambda i: (0, 0, 0)),
            pl.BlockSpec((1, HID), lambda i: (0, 0)),
            pl.BlockSpec((HID, NCLS), lambda i: (0, 0)),
            pl.BlockSpec((1, NCLS), lambda i: (0, 0)),
        ],
        out_specs=pl.BlockSpec((BB, NCLS), lambda i: (i, 0)),
        out_shape=jax.ShapeDtypeStruct((B, NCLS), jnp.float32),
    )(pooled, W1.reshape(3, D, HID), b1.reshape(1, HID), W2, b2.reshape(1, NCLS))


def kernel(tokens_0, tokens_1, tokens_2, emb_uni, emb_bi, emb_tri,
           W1, b1, W2, b2):
    pooled = _pooled_means(tokens_0, tokens_1, tokens_2,
                           emb_uni, emb_bi, emb_tri)
    return _mlp(pooled, W1, b1, W2, b2)
